# baseline (speedup 1.0000x reference)
import functools

import jax
import jax.numpy as jnp
from jax import lax
from jax.experimental import pallas as pl
from jax.experimental.pallas import tpu as pltpu
from jax.experimental.pallas import tpu_sc as plsc

_T = 32000
_TOUT = 8000
_HALF = _TOUT // 2
_B = 16
_L = 16
_NC = 2
_NS = 16
_K = 10

_TEN_LOG2 = 3.0102999566398120
_BIAS = 127.0 * _TEN_LOG2
_C = (4.886358058187659e-08, 1.442686777825966, -0.7211146144033768,
      0.47832354486771805, -0.34599601243320727, 0.23923166297195594,
      -0.13453425419770781, 0.05027750736969067, -0.008874696650988632)


def _edb10_biased(x):
    bits = lax.bitcast_convert_type(x, jnp.int32)
    e = lax.shift_right_logical(bits, 23).astype(jnp.float32)
    m = lax.bitcast_convert_type(
        (bits & jnp.int32(0x007FFFFF)) | jnp.int32(0x3F800000), jnp.float32)
    t = m - 1.0
    t2 = t * t
    t4 = t2 * t2
    lo = (_C[0] + _C[1] * t) + (_C[2] + _C[3] * t) * t2
    hi = (_C[4] + _C[5] * t) + (_C[6] + _C[7] * t) * t2
    p = lo + (hi + _C[8] * t4) * t4
    return (e + p) * _TEN_LOG2


def _edc_worker(h_ref, t_ref, out_ref, buf, en, pb, tv, ptv, ps,
                shared_en, shared_tv):
    c = lax.axis_index("c")
    s = lax.axis_index("s")
    wid = s * _NC + c
    arr = wid // _B
    row = wid % _B

    @pl.when(arr == 0)
    def _():
        pltpu.sync_copy(h_ref.at[row], buf)

    @pl.when(arr == 1)
    def _():
        pltpu.sync_copy(t_ref.at[row], buf)

    accs = tuple(jnp.zeros((_L,), jnp.float32) for _ in range(_K))

    def body1(i, accs):
        base = i * (2 * _K * _L)
        out = []
        for j in range(_K):
            v1 = buf[pl.ds(base + (2 * j) * _L, _L)]
            v2 = buf[pl.ds(base + (2 * j + 1) * _L, _L)]
            out.append(accs[j] + v1 * v1 + v2 * v2)
        return tuple(out)

    accs = lax.fori_loop(0, _T // (2 * _K * _L), body1, accs)
    acc = accs[0]
    for j in range(1, _K):
        acc = acc + accs[j]
    total = jnp.sum(acc)
    total_v = jnp.full((_L,), total, jnp.float32)
    tv[...] = total_v

    def body2(i, carry):
        base = i * (_K * _L)
        vs = [buf[pl.ds(base + j * _L, _L)] for j in range(_K)]
        pss = [v * v for v in vs]
        css = [plsc.cumsum(p) for p in pss]
        sums = [jnp.full((_L,), jnp.sum(p), jnp.float32) for p in pss]
        off = carry
        for j in range(_K):
            toff = total_v - off
            en[pl.ds(base + j * _L, _L)] = toff + (pss[j] - css[j])
            off = off + sums[j]
        return off

    lax.fori_loop(0, _TOUT // (_K * _L), body2, jnp.zeros((_L,), jnp.float32))

    pltpu.sync_copy(en, shared_en.at[pl.ds(s * _TOUT, _TOUT)])
    pltpu.sync_copy(tv, shared_tv.at[pl.ds(s * _L, _L)])
    plsc.subcore_barrier()

    partner = s ^ 8
    half_off = arr * _HALF
    pltpu.sync_copy(shared_en.at[pl.ds(partner * _TOUT + half_off, _HALF)], pb)
    pltpu.sync_copy(shared_tv.at[pl.ds(partner * _L, _L)], ptv)
    const_v = _edb10_biased(ptv[...]) - _edb10_biased(total_v) - _BIAS

    def body3(i, acc):
        base = i * (_K * _L)
        for j in range(_K):
            mine = en[pl.ds(half_off + base + j * _L, _L)]
            theirs = pb[pl.ds(base + j * _L, _L)]
            ok = (mine > 0) & (theirs > 0)
            y = _edb10_biased(mine / theirs)
            acc = acc + jnp.where(ok, jnp.abs(y + const_v), 0.0)
        return acc

    acc3 = lax.fori_loop(0, _HALF // (_K * _L), body3,
                         jnp.zeros((_L,), jnp.float32))
    zero_v = jnp.zeros((_L,), jnp.float32)
    for i in range(1, 8):
        ps[pl.ds(i * _L, _L)] = zero_v
    ps[pl.ds(0, _L)] = acc3
    pltpu.sync_copy(ps, out_ref.at[wid, 0])


@functools.partial(
    pl.kernel,
    out_type=jax.ShapeDtypeStruct((2 * _B, 1, 128), jnp.float32),
    mesh=plsc.VectorSubcoreMesh(core_axis_name="c", subcore_axis_name="s",
                                num_cores=_NC, num_subcores=_NS),
    compiler_params=pltpu.CompilerParams(needs_layout_passes=False),
    scratch_types=[
        pltpu.VMEM((_T,), jnp.float32),
        pltpu.VMEM((_TOUT,), jnp.float32),
        pltpu.VMEM((_HALF,), jnp.float32),
        pltpu.VMEM((_L,), jnp.float32),
        pltpu.VMEM((_L,), jnp.float32),
        pltpu.VMEM((128,), jnp.float32),
        pltpu.VMEM_SHARED((_NS * _TOUT,), jnp.float32),
        pltpu.VMEM_SHARED((_NS * _L,), jnp.float32),
    ],
)
def _edc_db_sc(h_ref, t_ref, out_ref, buf, en, pb, tv, ptv, ps,
               shared_en, shared_tv):
    _edc_worker(h_ref, t_ref, out_ref, buf, en, pb, tv, ptv, ps,
                shared_en, shared_tv)


def _l1_mean_body(p_ref, o_ref):
    o_ref[0, 0] = jnp.sum(p_ref[...]) * (1.0 / (_B * _TOUT))


def kernel(h, target_h):
    h2 = h.reshape(_B, _T)
    t2 = target_h.reshape(_B, _T)
    partials = _edc_db_sc(h2, t2)
    loss = pl.pallas_call(
        _l1_mean_body,
        out_shape=jax.ShapeDtypeStruct((1, 1), jnp.float32),
        out_specs=pl.BlockSpec(memory_space=pltpu.SMEM),
    )(partials)
    return loss[0, 0]


# speedup vs baseline: 1.0016x; 1.0016x over previous
import functools

import jax
import jax.numpy as jnp
from jax import lax
from jax.experimental import pallas as pl
from jax.experimental.pallas import tpu as pltpu
from jax.experimental.pallas import tpu_sc as plsc

_T = 32000
_TOUT = 8000
_HALF = _TOUT // 2
_B = 16
_L = 16
_NC = 2
_NS = 16
_K = 10

_TEN_LOG2 = 3.0102999566398120
_BIAS = 127.0 * _TEN_LOG2
_C = (4.886358058187659e-08, 1.442686777825966, -0.7211146144033768,
      0.47832354486771805, -0.34599601243320727, 0.23923166297195594,
      -0.13453425419770781, 0.05027750736969067, -0.008874696650988632)


def _edb10_biased(x):
    bits = lax.bitcast_convert_type(x, jnp.int32)
    e = lax.shift_right_logical(bits, 23).astype(jnp.float32)
    m = lax.bitcast_convert_type(
        (bits & jnp.int32(0x007FFFFF)) | jnp.int32(0x3F800000), jnp.float32)
    t = m - 1.0
    t2 = t * t
    t4 = t2 * t2
    lo = (_C[0] + _C[1] * t) + (_C[2] + _C[3] * t) * t2
    hi = (_C[4] + _C[5] * t) + (_C[6] + _C[7] * t) * t2
    p = lo + (hi + _C[8] * t4) * t4
    return (e + p) * _TEN_LOG2


def _edc_worker(h_ref, t_ref, out_ref, buf, en, pb, tv, ptv, ps,
                shared_en, shared_tv):
    c = lax.axis_index("c")
    s = lax.axis_index("s")
    wid = s * _NC + c
    arr = wid // _B
    row = wid % _B

    @pl.when(arr == 0)
    def _():
        pltpu.sync_copy(h_ref.at[row], buf)

    @pl.when(arr == 1)
    def _():
        pltpu.sync_copy(t_ref.at[row], buf)

    accs = tuple(jnp.zeros((_L,), jnp.float32) for _ in range(_K))

    def body1(i, accs):
        base = i * (2 * _K * _L)
        out = []
        for j in range(_K):
            v1 = buf[pl.ds(base + (2 * j) * _L, _L)]
            v2 = buf[pl.ds(base + (2 * j + 1) * _L, _L)]
            out.append(accs[j] + v1 * v1 + v2 * v2)
        return tuple(out)

    def shift_body1(i, accs):
        return body1(i + _TOUT // (2 * _K * _L), accs)

    accs = lax.fori_loop(0, (_T - _TOUT) // (2 * _K * _L), shift_body1, accs)
    acc = accs[0]
    for j in range(1, _K):
        acc = acc + accs[j]
    tail_total = jnp.sum(acc)

    def body2(i, carry):
        base = i * (_K * _L)
        vs = [buf[pl.ds(base + j * _L, _L)] for j in range(_K)]
        pss = [v * v for v in vs]
        css = [plsc.cumsum(p) for p in pss]
        sums = [jnp.full((_L,), jnp.sum(p), jnp.float32) for p in pss]
        off = carry
        for j in range(_K):
            en[pl.ds(base + j * _L, _L)] = off + (css[j] - pss[j])
            off = off + sums[j]
        return off

    head_total = lax.fori_loop(0, _TOUT // (_K * _L), body2,
                               jnp.zeros((_L,), jnp.float32))
    total_v = jnp.full((_L,), tail_total, jnp.float32) + head_total
    tv[...] = total_v

    pltpu.sync_copy(en, shared_en.at[pl.ds(s * _TOUT, _TOUT)])
    pltpu.sync_copy(tv, shared_tv.at[pl.ds(s * _L, _L)])
    plsc.subcore_barrier()

    partner = s ^ 8
    half_off = arr * _HALF
    pltpu.sync_copy(shared_en.at[pl.ds(partner * _TOUT + half_off, _HALF)], pb)
    pltpu.sync_copy(shared_tv.at[pl.ds(partner * _L, _L)], ptv)
    const_v = _edb10_biased(ptv[...]) - _edb10_biased(total_v) - _BIAS

    ptv_v = ptv[...]

    def body3(i, acc):
        base = i * (_K * _L)
        for j in range(_K):
            mine = total_v - en[pl.ds(half_off + base + j * _L, _L)]
            theirs = ptv_v - pb[pl.ds(base + j * _L, _L)]
            ok = (mine > 0) & (theirs > 0)
            y = _edb10_biased(mine / theirs)
            acc = acc + jnp.where(ok, jnp.abs(y + const_v), 0.0)
        return acc

    acc3 = lax.fori_loop(0, _HALF // (_K * _L), body3,
                         jnp.zeros((_L,), jnp.float32))
    zero_v = jnp.zeros((_L,), jnp.float32)
    for i in range(1, 8):
        ps[pl.ds(i * _L, _L)] = zero_v
    ps[pl.ds(0, _L)] = acc3
    pltpu.sync_copy(ps, out_ref.at[wid, 0])


@functools.partial(
    pl.kernel,
    out_type=jax.ShapeDtypeStruct((2 * _B, 1, 128), jnp.float32),
    mesh=plsc.VectorSubcoreMesh(core_axis_name="c", subcore_axis_name="s",
                                num_cores=_NC, num_subcores=_NS),
    compiler_params=pltpu.CompilerParams(needs_layout_passes=False),
    scratch_types=[
        pltpu.VMEM((_T,), jnp.float32),
        pltpu.VMEM((_TOUT,), jnp.float32),
        pltpu.VMEM((_HALF,), jnp.float32),
        pltpu.VMEM((_L,), jnp.float32),
        pltpu.VMEM((_L,), jnp.float32),
        pltpu.VMEM((128,), jnp.float32),
        pltpu.VMEM_SHARED((_NS * _TOUT,), jnp.float32),
        pltpu.VMEM_SHARED((_NS * _L,), jnp.float32),
    ],
)
def _edc_db_sc(h_ref, t_ref, out_ref, buf, en, pb, tv, ptv, ps,
               shared_en, shared_tv):
    _edc_worker(h_ref, t_ref, out_ref, buf, en, pb, tv, ptv, ps,
                shared_en, shared_tv)


def _l1_mean_body(p_ref, o_ref):
    o_ref[0, 0] = jnp.sum(p_ref[...]) * (1.0 / (_B * _TOUT))


def kernel(h, target_h):
    h2 = h.reshape(_B, _T)
    t2 = target_h.reshape(_B, _T)
    partials = _edc_db_sc(h2, t2)
    loss = pl.pallas_call(
        _l1_mean_body,
        out_shape=jax.ShapeDtypeStruct((1, 1), jnp.float32),
        out_specs=pl.BlockSpec(memory_space=pltpu.SMEM),
    )(partials)
    return loss[0, 0]
